# Initial kernel scaffold; baseline (speedup 1.0000x reference)
import jax
import jax.numpy as jnp
from jax.experimental import pallas as pl


def kernel(x, c1w, c1b, c2w, c2b, c3w, c3b, w1, w2, g1, b1, g2, b2):
    raise NotImplementedError("write your pallas kernel here")


# speedup vs baseline: 1.6772x; 1.6772x over previous
import numpy as np
import jax
import jax.numpy as jnp
from jax.experimental import pallas as pl

N = 128
DQ = 96
ROWS = N * DQ
ROWBLK = 1536


def _conv_stack_kernel(x1_ref, m1_ref, b1_ref, w2_ref, b2_ref, w3_ref,
                       b3_ref, out_ref):
    x1 = x1_ref[...]
    h1 = jnp.dot(x1, m1_ref[...], preferred_element_type=jnp.float32)
    h1 = jnp.maximum(h1 + b1_ref[...], 0.0)

    rid = jax.lax.broadcasted_iota(jnp.int32, (ROWBLK, 1), 0)
    d = rid % DQ
    z18 = jnp.zeros((1, 18), jnp.float32)
    h1_dn = jnp.where(d == 0, 0.0, jnp.concatenate([z18, h1[:-1, :]], axis=0))
    h1_up = jnp.where(d == DQ - 1, 0.0,
                      jnp.concatenate([h1[1:, :], z18], axis=0))
    a2 = jnp.concatenate([h1_dn, h1, h1_up], axis=1)
    h2 = jnp.dot(a2, w2_ref[...], preferred_element_type=jnp.float32)
    h2 = jnp.maximum(h2 + b2_ref[...], 0.0)

    z4 = jnp.zeros((1, 4), jnp.float32)
    h2_dn = jnp.where(d == 0, 0.0, jnp.concatenate([z4, h2[:-1, :]], axis=0))
    h2_up = jnp.where(d == DQ - 1, 0.0,
                      jnp.concatenate([h2[1:, :], z4], axis=0))
    a3 = jnp.concatenate([h2_dn, h2, h2_up], axis=1)
    h3 = jnp.dot(a3, w3_ref[...], preferred_element_type=jnp.float32)
    out_ref[...] = jnp.maximum(h3 + b3_ref[...], 0.0)


def _gcn_kernel(fr_ref, frt_ref, w1_ref, w2_ref, g1_ref, b1_ref, g2_ref,
                b2_ref, out_ref):
    fr = fr_ref[...]
    frt = frt_ref[...]
    gram = jnp.dot(fr, frt, preferred_element_type=jnp.float32, precision=jax.lax.Precision.HIGHEST)
    sqn = jnp.sum(fr * fr, axis=1, keepdims=True)
    sqn_t = jnp.sum(frt * frt, axis=0, keepdims=True)
    sq = sqn + sqn_t - 2.0 * gram
    e = jnp.exp(sq * (-1.0 / 10.0))

    colid = jax.lax.broadcasted_iota(jnp.int32, (N, N), 1)
    rowid = jax.lax.broadcasted_iota(jnp.int32, (N, N), 0)
    m = e
    selected = colid < 0
    for _ in range(N // 16):
        rmax = jnp.max(m, axis=1, keepdims=True)
        first = jnp.min(jnp.where(m >= rmax, colid, N), axis=1, keepdims=True)
        sel = colid == first
        selected = jnp.logical_or(selected, sel)
        m = jnp.where(sel, -1.0, m)

    adj0 = jnp.where(selected, 0.9, 0.0)
    adj0 = jnp.where(rowid == colid, 0.5, adj0)
    eye = jnp.where(rowid == colid, 1.0, 0.0)
    adjacency = eye + adj0 * e
    invd = 1.0 / jnp.sqrt(jnp.sum(adjacency, axis=1, keepdims=True) + 1.0)
    dinv = jnp.where(rowid == colid, invd, 0.0)
    anorm = jnp.dot(jnp.dot(dinv, adjacency, preferred_element_type=jnp.float32),
                    dinv, preferred_element_type=jnp.float32)

    u1 = jnp.dot(fr, w1_ref[...], preferred_element_type=jnp.float32)
    s1 = jnp.dot(anorm, u1, preferred_element_type=jnp.float32)
    mu1 = jnp.mean(s1, axis=0, keepdims=True)
    xc1 = s1 - mu1
    var1 = jnp.mean(xc1 * xc1, axis=0, keepdims=True)
    s1 = jnp.tanh(g1_ref[...] * xc1 * jax.lax.rsqrt(var1 + 1e-5) + b1_ref[...])

    u2 = jnp.dot(s1, w2_ref[...], preferred_element_type=jnp.float32)
    s2 = jnp.dot(anorm, u2, preferred_element_type=jnp.float32)
    mu2 = jnp.mean(s2, axis=0, keepdims=True)
    xc2 = s2 - mu2
    var2 = jnp.mean(xc2 * xc2, axis=0, keepdims=True)
    out_ref[...] = jnp.tanh(g2_ref[...] * xc2 * jax.lax.rsqrt(var2 + 1e-5)
                            + b2_ref[...])


def _conv1_matrix(c1w):
    w = jnp.transpose(c1w[:, 0], (1, 2, 3, 0))
    vals = jnp.broadcast_to(w[:, :, :, :, None, None], (7, 3, 3, 2, 3, 3))
    kd, kh, kw, o, ho, wo = np.meshgrid(
        np.arange(7), np.arange(3), np.arange(3), np.arange(2),
        np.arange(3), np.arange(3), indexing="ij")
    rows = (kd * 25 + (ho + kh) * 5 + (wo + kw)).ravel()
    cols = (o * 9 + ho * 3 + wo).ravel()
    return jnp.zeros((175, 18), jnp.float32).at[rows, cols].set(vals.ravel())


def kernel(x, c1w, c1b, c2w, c2b, c3w, c3b, w1, w2, g1, b1, g2, b2):
    xp = jnp.pad(x.reshape(N, 100, 25), ((0, 0), (1, 1), (0, 0)))
    x1 = jnp.concatenate([xp[:, k:k + DQ, :] for k in range(7)],
                         axis=2).reshape(ROWS, 175)
    m1 = _conv1_matrix(c1w)
    b1c = jnp.repeat(c1b, 9)[None, :]
    w2cat = jnp.transpose(c2w, (2, 1, 3, 4, 0)).reshape(54, 4)
    w3cat = jnp.transpose(c3w[:, :, :, 0, 0], (2, 1, 0)).reshape(12, 16)
    b2c = c2b[None, :]
    b3c = c3b[None, :]

    h3 = pl.pallas_call(
        _conv_stack_kernel,
        grid=(ROWS // ROWBLK,),
        in_specs=[
            pl.BlockSpec((ROWBLK, 175), lambda i: (i, 0)),
            pl.BlockSpec((175, 18), lambda i: (0, 0)),
            pl.BlockSpec((1, 18), lambda i: (0, 0)),
            pl.BlockSpec((54, 4), lambda i: (0, 0)),
            pl.BlockSpec((1, 4), lambda i: (0, 0)),
            pl.BlockSpec((12, 16), lambda i: (0, 0)),
            pl.BlockSpec((1, 16), lambda i: (0, 0)),
        ],
        out_specs=pl.BlockSpec((ROWBLK, 16), lambda i: (i, 0)),
        out_shape=jax.ShapeDtypeStruct((ROWS, 16), jnp.float32),
    )(x1, m1, b1c, w2cat, b2c, w3cat, b3c)

    fr = h3.reshape(N, DQ * 16)
    frt = fr.T
    w1p = w1.reshape(16, DQ, N).transpose(1, 0, 2).reshape(DQ * 16, N)

    gcn_in = (fr, frt, w1p, w2, g1[None, :], b1[None, :],
              g2[None, :], b2[None, :])
    return pl.pallas_call(
        _gcn_kernel,
        in_specs=[pl.BlockSpec(a.shape, lambda s=a.shape: (0,) * len(s))
                  for a in gcn_in],
        out_specs=pl.BlockSpec((N, 16), lambda: (0, 0)),
        out_shape=jax.ShapeDtypeStruct((N, 16), jnp.float32),
    )(*gcn_in)


# speedup vs baseline: 2.3546x; 1.4039x over previous
import numpy as np
import jax
import jax.numpy as jnp
from jax.experimental import pallas as pl

N = 128
DQ = 96
BN = 16
RB = BN * DQ


def _conv_stack_kernel(x_ref, m1_ref, b1_ref, w2_ref, b2_ref, w3_ref,
                       b3_ref, out_ref):
    xv = x_ref[...]
    zpad = jnp.zeros((BN, 1, 25), jnp.float32)
    xp = jnp.concatenate([zpad, xv, zpad], axis=1)
    x1 = jnp.concatenate([xp[:, k:k + DQ, :] for k in range(7)],
                         axis=2).reshape(RB, 175)
    h1 = jnp.dot(x1, m1_ref[...], preferred_element_type=jnp.float32)
    h1 = jnp.maximum(h1 + b1_ref[...], 0.0)

    rid = jax.lax.broadcasted_iota(jnp.int32, (RB, 1), 0)
    d = rid % DQ
    z18 = jnp.zeros((1, 18), jnp.float32)
    h1_dn = jnp.where(d == 0, 0.0, jnp.concatenate([z18, h1[:-1, :]], axis=0))
    h1_up = jnp.where(d == DQ - 1, 0.0,
                      jnp.concatenate([h1[1:, :], z18], axis=0))
    a2 = jnp.concatenate([h1_dn, h1, h1_up], axis=1)
    h2 = jnp.dot(a2, w2_ref[...], preferred_element_type=jnp.float32)
    h2 = jnp.maximum(h2 + b2_ref[...], 0.0)

    z4 = jnp.zeros((1, 4), jnp.float32)
    h2_dn = jnp.where(d == 0, 0.0, jnp.concatenate([z4, h2[:-1, :]], axis=0))
    h2_up = jnp.where(d == DQ - 1, 0.0,
                      jnp.concatenate([h2[1:, :], z4], axis=0))
    a3 = jnp.concatenate([h2_dn, h2, h2_up], axis=1)
    h3 = jnp.dot(a3, w3_ref[...], preferred_element_type=jnp.float32)
    out_ref[...] = jnp.maximum(h3 + b3_ref[...], 0.0)


def _gcn_kernel(fr_ref, w1_ref, w2_ref, g1_ref, b1_ref, g2_ref,
                b2_ref, out_ref):
    fr = fr_ref[...]
    gram = jax.lax.dot_general(
        fr, fr, (((1,), (1,)), ((), ())),
        preferred_element_type=jnp.float32,
        precision=jax.lax.Precision.HIGHEST)
    colid = jax.lax.broadcasted_iota(jnp.int32, (N, N), 1)
    rowid = jax.lax.broadcasted_iota(jnp.int32, (N, N), 0)
    diag = jnp.where(rowid == colid, gram, 0.0)
    sqn = jnp.sum(diag, axis=1, keepdims=True)
    sqn_t = jnp.sum(diag, axis=0, keepdims=True)
    sq = sqn + sqn_t - 2.0 * gram
    e = jnp.exp(sq * (-1.0 / 10.0))

    m = e
    selected = colid < 0
    for _ in range(N // 16):
        rmax = jnp.max(m, axis=1, keepdims=True)
        first = jnp.min(jnp.where(m >= rmax, colid, N), axis=1, keepdims=True)
        sel = colid == first
        selected = jnp.logical_or(selected, sel)
        m = jnp.where(sel, -1.0, m)

    adj0 = jnp.where(selected, 0.9, 0.0)
    adj0 = jnp.where(rowid == colid, 0.5, adj0)
    eye = jnp.where(rowid == colid, 1.0, 0.0)
    adjacency = eye + adj0 * e
    invd = 1.0 / jnp.sqrt(jnp.sum(adjacency, axis=1, keepdims=True) + 1.0)
    dinv = jnp.where(rowid == colid, invd, 0.0)
    anorm = jnp.dot(jnp.dot(dinv, adjacency, preferred_element_type=jnp.float32),
                    dinv, preferred_element_type=jnp.float32)

    u1 = jnp.dot(fr, w1_ref[...], preferred_element_type=jnp.float32)
    s1 = jnp.dot(anorm, u1, preferred_element_type=jnp.float32)
    mu1 = jnp.mean(s1, axis=0, keepdims=True)
    xc1 = s1 - mu1
    var1 = jnp.mean(xc1 * xc1, axis=0, keepdims=True)
    s1 = jnp.tanh(g1_ref[...] * xc1 * jax.lax.rsqrt(var1 + 1e-5) + b1_ref[...])

    u2 = jnp.dot(s1, w2_ref[...], preferred_element_type=jnp.float32)
    s2 = jnp.dot(anorm, u2, preferred_element_type=jnp.float32)
    mu2 = jnp.mean(s2, axis=0, keepdims=True)
    xc2 = s2 - mu2
    var2 = jnp.mean(xc2 * xc2, axis=0, keepdims=True)
    out_ref[...] = jnp.tanh(g2_ref[...] * xc2 * jax.lax.rsqrt(var2 + 1e-5)
                            + b2_ref[...])


def _conv1_matrix(c1w):
    w = jnp.transpose(c1w[:, 0], (1, 2, 3, 0))
    vals = jnp.broadcast_to(w[:, :, :, :, None, None],
                            (7, 3, 3, 2, 3, 3)).reshape(-1)
    kd, kh, kw, o, ho, wo = np.meshgrid(
        np.arange(7), np.arange(3), np.arange(3), np.arange(2),
        np.arange(3), np.arange(3), indexing="ij")
    rows = (kd * 25 + (ho + kh) * 5 + (wo + kw)).ravel()
    cols = (o * 9 + ho * 3 + wo).ravel()
    r_onehot = np.zeros((1134, 175), np.float32)
    r_onehot[np.arange(1134), rows] = 1.0
    c_onehot = np.zeros((1134, 18), np.float32)
    c_onehot[np.arange(1134), cols] = 1.0
    tmp = vals[:, None] * jnp.asarray(c_onehot)
    return jax.lax.dot_general(
        jnp.asarray(r_onehot), tmp, (((0,), (0,)), ((), ())),
        precision=jax.lax.Precision.HIGHEST)


def kernel(x, c1w, c1b, c2w, c2b, c3w, c3b, w1, w2, g1, b1, g2, b2):
    xr = x.reshape(N, 100, 25)
    m1 = _conv1_matrix(c1w)
    b1c = jnp.repeat(c1b, 9)[None, :]
    w2cat = jnp.transpose(c2w, (2, 1, 3, 4, 0)).reshape(54, 4)
    w3cat = jnp.transpose(c3w[:, :, :, 0, 0], (2, 1, 0)).reshape(12, 16)
    b2c = c2b[None, :]
    b3c = c3b[None, :]

    h3 = pl.pallas_call(
        _conv_stack_kernel,
        grid=(N // BN,),
        in_specs=[
            pl.BlockSpec((BN, 100, 25), lambda i: (i, 0, 0)),
            pl.BlockSpec((175, 18), lambda i: (0, 0)),
            pl.BlockSpec((1, 18), lambda i: (0, 0)),
            pl.BlockSpec((54, 4), lambda i: (0, 0)),
            pl.BlockSpec((1, 4), lambda i: (0, 0)),
            pl.BlockSpec((12, 16), lambda i: (0, 0)),
            pl.BlockSpec((1, 16), lambda i: (0, 0)),
        ],
        out_specs=pl.BlockSpec((RB, 16), lambda i: (i, 0)),
        out_shape=jax.ShapeDtypeStruct((N * DQ, 16), jnp.float32),
    )(xr, m1, b1c, w2cat, b2c, w3cat, b3c)

    fr = h3.reshape(N, DQ * 16)
    w1p = w1.reshape(16, DQ, N).transpose(1, 0, 2).reshape(DQ * 16, N)

    gcn_in = (fr, w1p, w2, g1[None, :], b1[None, :], g2[None, :], b2[None, :])
    return pl.pallas_call(
        _gcn_kernel,
        in_specs=[pl.BlockSpec(a.shape, lambda s=a.shape: (0,) * len(s))
                  for a in gcn_in],
        out_specs=pl.BlockSpec((N, 16), lambda: (0, 0)),
        out_shape=jax.ShapeDtypeStruct((N, 16), jnp.float32),
    )(*gcn_in)


# speedup vs baseline: 2.4334x; 1.0334x over previous
import numpy as np
import jax
import jax.numpy as jnp
from jax.experimental import pallas as pl

N = 128
DQ = 96
BN = 16
RB = BN * DQ


def _conv_stack_kernel(x_ref, m1_ref, b1_ref, w2_ref, b2_ref, w3_ref,
                       b3_ref, out_ref):
    xv = x_ref[...]
    zpad = jnp.zeros((BN, 1, 25), jnp.float32)
    zpad3 = jnp.zeros((BN, 3, 25), jnp.float32)
    xp = jnp.concatenate([zpad, xv, zpad3], axis=1)
    x1 = jnp.concatenate([xp[:, k:k + DQ, :] for k in range(7)],
                         axis=2).reshape(RB, 175)
    h1 = jnp.dot(x1, m1_ref[...], preferred_element_type=jnp.float32)
    h1 = jnp.maximum(h1 + b1_ref[...], 0.0)

    rid = jax.lax.broadcasted_iota(jnp.int32, (RB, 1), 0)
    d = rid % DQ
    z18 = jnp.zeros((1, 18), jnp.float32)
    h1_dn = jnp.where(d == 0, 0.0, jnp.concatenate([z18, h1[:-1, :]], axis=0))
    h1_up = jnp.where(d == DQ - 1, 0.0,
                      jnp.concatenate([h1[1:, :], z18], axis=0))
    a2 = jnp.concatenate([h1_dn, h1, h1_up], axis=1)
    h2 = jnp.dot(a2, w2_ref[...], preferred_element_type=jnp.float32)
    h2 = jnp.maximum(h2 + b2_ref[...], 0.0)

    z4 = jnp.zeros((1, 4), jnp.float32)
    h2_dn = jnp.where(d == 0, 0.0, jnp.concatenate([z4, h2[:-1, :]], axis=0))
    h2_up = jnp.where(d == DQ - 1, 0.0,
                      jnp.concatenate([h2[1:, :], z4], axis=0))
    a3 = jnp.concatenate([h2_dn, h2, h2_up], axis=1)
    h3 = jnp.dot(a3, w3_ref[...], preferred_element_type=jnp.float32)
    out_ref[...] = jnp.maximum(h3 + b3_ref[...], 0.0)


def _gcn_kernel(fr_ref, w1_ref, w2_ref, g1_ref, b1_ref, g2_ref,
                b2_ref, out_ref):
    fr = fr_ref[...]
    gram = jax.lax.dot_general(
        fr, fr, (((1,), (1,)), ((), ())),
        preferred_element_type=jnp.float32,
        precision=jax.lax.Precision.HIGHEST)
    colid = jax.lax.broadcasted_iota(jnp.int32, (N, N), 1)
    rowid = jax.lax.broadcasted_iota(jnp.int32, (N, N), 0)
    diag = jnp.where(rowid == colid, gram, 0.0)
    sqn = jnp.sum(diag, axis=1, keepdims=True)
    sqn_t = jnp.sum(diag, axis=0, keepdims=True)
    sq = sqn + sqn_t - 2.0 * gram
    e = jnp.exp(sq * (-1.0 / 10.0))

    m = e
    selected = colid < 0
    for _ in range(N // 16):
        rmax = jnp.max(m, axis=1, keepdims=True)
        first = jnp.min(jnp.where(m >= rmax, colid, N), axis=1, keepdims=True)
        sel = colid == first
        selected = jnp.logical_or(selected, sel)
        m = jnp.where(sel, -1.0, m)

    adj0 = jnp.where(selected, 0.9, 0.0)
    adj0 = jnp.where(rowid == colid, 0.5, adj0)
    eye = jnp.where(rowid == colid, 1.0, 0.0)
    adjacency = eye + adj0 * e
    invd = 1.0 / jnp.sqrt(jnp.sum(adjacency, axis=1, keepdims=True) + 1.0)
    dinv = jnp.where(rowid == colid, invd, 0.0)
    anorm = jnp.dot(jnp.dot(dinv, adjacency, preferred_element_type=jnp.float32),
                    dinv, preferred_element_type=jnp.float32)

    w1p = jnp.transpose(w1_ref[...].reshape(16, DQ, N),
                        (1, 0, 2)).reshape(DQ * 16, N)
    u1 = jnp.dot(fr, w1p, preferred_element_type=jnp.float32)
    s1 = jnp.dot(anorm, u1, preferred_element_type=jnp.float32)
    mu1 = jnp.mean(s1, axis=0, keepdims=True)
    xc1 = s1 - mu1
    var1 = jnp.mean(xc1 * xc1, axis=0, keepdims=True)
    s1 = jnp.tanh(g1_ref[...] * xc1 * jax.lax.rsqrt(var1 + 1e-5) + b1_ref[...])

    u2 = jnp.dot(s1, w2_ref[...], preferred_element_type=jnp.float32)
    s2 = jnp.dot(anorm, u2, preferred_element_type=jnp.float32)
    mu2 = jnp.mean(s2, axis=0, keepdims=True)
    xc2 = s2 - mu2
    var2 = jnp.mean(xc2 * xc2, axis=0, keepdims=True)
    out_ref[...] = jnp.tanh(g2_ref[...] * xc2 * jax.lax.rsqrt(var2 + 1e-5)
                            + b2_ref[...])


def _placement_constants():
    o, kd, kh, kw, ho, wo = np.meshgrid(
        np.arange(2), np.arange(7), np.arange(3), np.arange(3),
        np.arange(3), np.arange(3), indexing="ij")
    rows = (kd * 25 + (ho + kh) * 5 + (wo + kw)).ravel()
    cols = (o * 9 + ho * 3 + wo).ravel()
    fidx = (o * 63 + kd * 9 + kh * 3 + kw).ravel()
    e1 = np.zeros((1134, 126), np.float32)
    e1[np.arange(1134), fidx] = 1.0
    r1t = np.zeros((175, 1134), np.float32)
    r1t[rows, np.arange(1134)] = 1.0
    c1oh = np.zeros((1134, 18), np.float32)
    c1oh[np.arange(1134), cols] = 1.0
    c, i, kd, kh, kw = np.meshgrid(
        np.arange(4), np.arange(2), np.arange(3), np.arange(3), np.arange(3),
        indexing="ij")
    rows2 = (kd * 18 + i * 9 + kh * 3 + kw).ravel()
    r2t = np.zeros((54, 216), np.float32)
    r2t[rows2, np.arange(216)] = 1.0
    c2oh = np.zeros((216, 4), np.float32)
    c2oh[np.arange(216), c.ravel()] = 1.0
    c, i, kd = np.meshgrid(np.arange(16), np.arange(4), np.arange(3),
                           indexing="ij")
    rows3 = (kd * 4 + i).ravel()
    r3t = np.zeros((12, 192), np.float32)
    r3t[rows3, np.arange(192)] = 1.0
    c3oh = np.zeros((192, 16), np.float32)
    c3oh[np.arange(192), c.ravel()] = 1.0
    b1sel = np.zeros((2, 18), np.float32)
    b1sel[np.repeat(np.arange(2), 9), np.arange(18)] = 1.0
    return e1, r1t, c1oh, r2t, c2oh, r3t, c3oh, b1sel


_E1, _R1T, _C1OH, _R2T, _C2OH, _R3T, _C3OH, _B1SEL = _placement_constants()
_HI = jax.lax.Precision.HIGHEST


def _weight_prep(c1wf, c1br, c2wf, c3wf, e1, r1t, c1oh, r2t, c2oh, r3t,
                 c3oh, b1sel):
    vals1 = jax.lax.dot_general(e1, c1wf, (((1,), (0,)), ((), ())))
    m1 = jax.lax.dot_general(r1t, vals1 * c1oh,
                             (((1,), (0,)), ((), ())))
    b1c = jax.lax.dot_general(c1br, b1sel, (((1,), (0,)), ((), ())),
                              precision=_HI)
    w2cat = jax.lax.dot_general(r2t, c2wf * c2oh,
                                (((1,), (0,)), ((), ())))
    w3cat = jax.lax.dot_general(r3t, c3wf * c3oh,
                                (((1,), (0,)), ((), ())))
    return m1, b1c, w2cat, w3cat


def _conv_stack_kernel2(x_ref, c1wf_ref, c1br_ref, c2wf_ref, b2_ref,
                        c3wf_ref, b3_ref, e1_ref, r1t_ref, c1oh_ref,
                        r2t_ref, c2oh_ref, r3t_ref, c3oh_ref, b1sel_ref,
                        out_ref):
    m1, b1c, w2cat, w3cat = _weight_prep(
        c1wf_ref[...], c1br_ref[...], c2wf_ref[...], c3wf_ref[...],
        e1_ref[...], r1t_ref[...], c1oh_ref[...], r2t_ref[...],
        c2oh_ref[...], r3t_ref[...], c3oh_ref[...], b1sel_ref[...])

    xv = x_ref[...]
    zpad = jnp.zeros((BN, 1, 25), jnp.float32)
    zpad3 = jnp.zeros((BN, 3, 25), jnp.float32)
    xp = jnp.concatenate([zpad, xv, zpad3], axis=1)
    x1 = jnp.concatenate([xp[:, k:k + DQ, :] for k in range(7)],
                         axis=2).reshape(RB, 175)
    h1 = jnp.dot(x1, m1, preferred_element_type=jnp.float32)
    h1 = jnp.maximum(h1 + b1c, 0.0)

    rid = jax.lax.broadcasted_iota(jnp.int32, (RB, 1), 0)
    d = rid % DQ
    z18 = jnp.zeros((1, 18), jnp.float32)
    h1_dn = jnp.where(d == 0, 0.0, jnp.concatenate([z18, h1[:-1, :]], axis=0))
    h1_up = jnp.where(d == DQ - 1, 0.0,
                      jnp.concatenate([h1[1:, :], z18], axis=0))
    a2 = jnp.concatenate([h1_dn, h1, h1_up], axis=1)
    h2 = jnp.dot(a2, w2cat, preferred_element_type=jnp.float32)
    h2 = jnp.maximum(h2 + b2_ref[...], 0.0)

    z4 = jnp.zeros((1, 4), jnp.float32)
    h2_dn = jnp.where(d == 0, 0.0, jnp.concatenate([z4, h2[:-1, :]], axis=0))
    h2_up = jnp.where(d == DQ - 1, 0.0,
                      jnp.concatenate([h2[1:, :], z4], axis=0))
    a3 = jnp.concatenate([h2_dn, h2, h2_up], axis=1)
    h3 = jnp.dot(a3, w3cat, preferred_element_type=jnp.float32)
    out_ref[...] = jnp.maximum(h3 + b3_ref[...], 0.0)


def kernel(x, c1w, c1b, c2w, c2b, c3w, c3b, w1, w2, g1, b1, g2, b2):
    xr = x.reshape(N, 100, 25)
    conv_in = (xr, c1w.reshape(126, 1), c1b[None, :], c2w.reshape(216, 1),
               c2b[None, :], c3w.reshape(192, 1), c3b[None, :],
               jnp.asarray(_E1), jnp.asarray(_R1T), jnp.asarray(_C1OH),
               jnp.asarray(_R2T), jnp.asarray(_C2OH), jnp.asarray(_R3T),
               jnp.asarray(_C3OH), jnp.asarray(_B1SEL))
    conv_specs = [pl.BlockSpec((BN, 100, 25), lambda i: (i, 0, 0))]
    conv_specs += [pl.BlockSpec(a.shape, lambda i, s=a.shape: (0,) * len(s))
                   for a in conv_in[1:]]

    h3 = pl.pallas_call(
        _conv_stack_kernel2,
        grid=(N // BN,),
        in_specs=conv_specs,
        out_specs=pl.BlockSpec((RB, 16), lambda i: (i, 0)),
        out_shape=jax.ShapeDtypeStruct((N * DQ, 16), jnp.float32),
    )(*conv_in)

    fr = h3.reshape(N, DQ * 16)
    gcn_in = (fr, w1, w2, g1[None, :], b1[None, :], g2[None, :], b2[None, :])
    return pl.pallas_call(
        _gcn_kernel,
        in_specs=[pl.BlockSpec(a.shape, lambda s=a.shape: (0,) * len(s))
                  for a in gcn_in],
        out_specs=pl.BlockSpec((N, 16), lambda: (0, 0)),
        out_shape=jax.ShapeDtypeStruct((N, 16), jnp.float32),
    )(*gcn_in)


# speedup vs baseline: 3.2961x; 1.3546x over previous
import numpy as np
import jax
import jax.numpy as jnp
from jax.experimental import pallas as pl

N = 128
DQ = 96
RB = N * DQ


def _placement_constants():
    o, kd, kh, kw, ho, wo = np.meshgrid(
        np.arange(2), np.arange(7), np.arange(3), np.arange(3),
        np.arange(3), np.arange(3), indexing="ij")
    rows = (kd * 25 + (ho + kh) * 5 + (wo + kw)).ravel()
    cols = (o * 9 + ho * 3 + wo).ravel()
    fidx = (o * 63 + kd * 9 + kh * 3 + kw).ravel()
    e1 = np.zeros((1134, 126), np.float32)
    e1[np.arange(1134), fidx] = 1.0
    r1t = np.zeros((175, 1134), np.float32)
    r1t[rows, np.arange(1134)] = 1.0
    c1oh = np.zeros((1134, 18), np.float32)
    c1oh[np.arange(1134), cols] = 1.0
    c, i, kd, kh, kw = np.meshgrid(
        np.arange(4), np.arange(2), np.arange(3), np.arange(3), np.arange(3),
        indexing="ij")
    rows2 = (kd * 18 + i * 9 + kh * 3 + kw).ravel()
    r2t = np.zeros((54, 216), np.float32)
    r2t[rows2, np.arange(216)] = 1.0
    c2oh = np.zeros((216, 4), np.float32)
    c2oh[np.arange(216), c.ravel()] = 1.0
    c, i, kd = np.meshgrid(np.arange(16), np.arange(4), np.arange(3),
                           indexing="ij")
    rows3 = (kd * 4 + i).ravel()
    r3t = np.zeros((12, 192), np.float32)
    r3t[rows3, np.arange(192)] = 1.0
    c3oh = np.zeros((192, 16), np.float32)
    c3oh[np.arange(192), c.ravel()] = 1.0
    b1sel = np.zeros((2, 18), np.float32)
    b1sel[np.repeat(np.arange(2), 9), np.arange(18)] = 1.0
    return e1, r1t, c1oh, r2t, c2oh, r3t, c3oh, b1sel


_E1, _R1T, _C1OH, _R2T, _C2OH, _R3T, _C3OH, _B1SEL = _placement_constants()
_HI = jax.lax.Precision.HIGHEST


def _weight_prep(c1wf, c1br, c2wf, c3wf, e1, r1t, c1oh, r2t, c2oh, r3t,
                 c3oh, b1sel):
    vals1 = jax.lax.dot_general(e1, c1wf, (((1,), (0,)), ((), ())))
    m1 = jax.lax.dot_general(r1t, vals1 * c1oh,
                             (((1,), (0,)), ((), ())))
    b1c = jax.lax.dot_general(c1br, b1sel, (((1,), (0,)), ((), ())),
                              precision=_HI)
    w2cat = jax.lax.dot_general(r2t, c2wf * c2oh,
                                (((1,), (0,)), ((), ())))
    w3cat = jax.lax.dot_general(r3t, c3wf * c3oh,
                                (((1,), (0,)), ((), ())))
    return m1, b1c, w2cat, w3cat


def _fused_kernel(x_ref, c1wf_ref, c1br_ref, c2wf_ref, cb2_ref,
                  c3wf_ref, cb3_ref, e1_ref, r1t_ref, c1oh_ref,
                  r2t_ref, c2oh_ref, r3t_ref, c3oh_ref, b1sel_ref,
                  w1_ref, w2_ref, g1_ref, gb1_ref, g2_ref, gb2_ref,
                  out_ref):
    m1, b1c, w2cat, w3cat = _weight_prep(
        c1wf_ref[...], c1br_ref[...], c2wf_ref[...], c3wf_ref[...],
        e1_ref[...], r1t_ref[...], c1oh_ref[...], r2t_ref[...],
        c2oh_ref[...], r3t_ref[...], c3oh_ref[...], b1sel_ref[...])

    xt = jnp.transpose(x_ref[...], (1, 0, 2))
    zpad = jnp.zeros((1, N, 25), jnp.float32)
    zpad3 = jnp.zeros((3, N, 25), jnp.float32)
    xpd = jnp.concatenate([zpad, xt, zpad3], axis=0)
    x1 = jnp.concatenate([xpd[k:k + DQ] for k in range(7)],
                         axis=2).reshape(RB, 175)
    h1 = jnp.dot(x1, m1, preferred_element_type=jnp.float32)
    h1 = jnp.maximum(h1 + b1c, 0.0)

    z18 = jnp.zeros((N, 18), jnp.float32)
    a2 = jnp.concatenate(
        [jnp.concatenate([z18, h1[:-N, :]], axis=0), h1,
         jnp.concatenate([h1[N:, :], z18], axis=0)], axis=1)
    h2 = jnp.dot(a2, w2cat, preferred_element_type=jnp.float32)
    h2 = jnp.maximum(h2 + cb2_ref[...], 0.0)

    z4 = jnp.zeros((N, 4), jnp.float32)
    a3 = jnp.concatenate(
        [jnp.concatenate([z4, h2[:-N, :]], axis=0), h2,
         jnp.concatenate([h2[N:, :], z4], axis=0)], axis=1)
    h3 = jnp.dot(a3, w3cat, preferred_element_type=jnp.float32)
    h3 = jnp.maximum(h3 + cb3_ref[...], 0.0)

    fr = jnp.concatenate([h3[dd * N:(dd + 1) * N, :] for dd in range(DQ)],
                         axis=1)

    gram = jax.lax.dot_general(
        fr, fr, (((1,), (1,)), ((), ())),
        preferred_element_type=jnp.float32,
        precision=jax.lax.Precision.HIGHEST)
    colid = jax.lax.broadcasted_iota(jnp.int32, (N, N), 1)
    rowid = jax.lax.broadcasted_iota(jnp.int32, (N, N), 0)
    diag = jnp.where(rowid == colid, gram, 0.0)
    sqn = jnp.sum(diag, axis=1, keepdims=True)
    sqn_t = jnp.sum(diag, axis=0, keepdims=True)
    sq = sqn + sqn_t - 2.0 * gram
    e = jnp.exp(sq * (-1.0 / 10.0))

    m = e
    selected = colid < 0
    for _ in range(8):
        rmax = jnp.max(m, axis=1, keepdims=True)
        first = jnp.min(jnp.where(m >= rmax, colid, N), axis=1, keepdims=True)
        sel = colid == first
        selected = jnp.logical_or(selected, sel)
        m = jnp.where(sel, -1.0, m)

    adj0 = jnp.where(selected, 0.9, 0.0)
    adj0 = jnp.where(rowid == colid, 0.5, adj0)
    eye = jnp.where(rowid == colid, 1.0, 0.0)
    adjacency = eye + adj0 * e
    invd = 1.0 / jnp.sqrt(jnp.sum(adjacency, axis=1, keepdims=True) + 1.0)
    dinv = jnp.where(rowid == colid, invd, 0.0)
    anorm = jnp.dot(jnp.dot(dinv, adjacency, preferred_element_type=jnp.float32),
                    dinv, preferred_element_type=jnp.float32)

    w1p = jnp.transpose(w1_ref[...].reshape(16, DQ, N),
                        (1, 0, 2)).reshape(DQ * 16, N)
    u1 = jnp.dot(fr, w1p, preferred_element_type=jnp.float32)
    s1 = jnp.dot(anorm, u1, preferred_element_type=jnp.float32)
    mu1 = jnp.mean(s1, axis=0, keepdims=True)
    xc1 = s1 - mu1
    var1 = jnp.mean(xc1 * xc1, axis=0, keepdims=True)
    s1 = jnp.tanh(g1_ref[...] * xc1 * jax.lax.rsqrt(var1 + 1e-5) + gb1_ref[...])

    u2 = jnp.dot(s1, w2_ref[...], preferred_element_type=jnp.float32)
    s2 = jnp.dot(anorm, u2, preferred_element_type=jnp.float32)
    mu2 = jnp.mean(s2, axis=0, keepdims=True)
    xc2 = s2 - mu2
    var2 = jnp.mean(xc2 * xc2, axis=0, keepdims=True)
    out_ref[...] = jnp.tanh(g2_ref[...] * xc2 * jax.lax.rsqrt(var2 + 1e-5)
                            + gb2_ref[...])


def kernel(x, c1w, c1b, c2w, c2b, c3w, c3b, w1, w2, g1, b1, g2, b2):
    ins = (x.reshape(N, 100, 25), c1w.reshape(126, 1), c1b[None, :],
           c2w.reshape(216, 1), c2b[None, :], c3w.reshape(192, 1),
           c3b[None, :], jnp.asarray(_E1), jnp.asarray(_R1T),
           jnp.asarray(_C1OH), jnp.asarray(_R2T), jnp.asarray(_C2OH),
           jnp.asarray(_R3T), jnp.asarray(_C3OH), jnp.asarray(_B1SEL),
           w1, w2, g1[None, :], b1[None, :], g2[None, :], b2[None, :])
    return pl.pallas_call(
        _fused_kernel,
        in_specs=[pl.BlockSpec(a.shape, lambda s=a.shape: (0,) * len(s))
                  for a in ins],
        out_specs=pl.BlockSpec((N, 16), lambda: (0, 0)),
        out_shape=jax.ShapeDtypeStruct((N, 16), jnp.float32),
    )(*ins)


# speedup vs baseline: 3.4373x; 1.0428x over previous
import numpy as np
import jax
import jax.numpy as jnp
from jax.experimental import pallas as pl

N = 128
DQ = 96
RB = N * DQ


def _placement_constants():
    o, kd, kh, kw, ho, wo = np.meshgrid(
        np.arange(2), np.arange(7), np.arange(3), np.arange(3),
        np.arange(3), np.arange(3), indexing="ij")
    rows = (kd * 25 + (ho + kh) * 5 + (wo + kw)).ravel()
    cols = (o * 9 + ho * 3 + wo).ravel()
    fidx = (o * 63 + kd * 9 + kh * 3 + kw).ravel()
    e1 = np.zeros((1134, 126), np.float32)
    e1[np.arange(1134), fidx] = 1.0
    r1t = np.zeros((175, 1134), np.float32)
    r1t[rows, np.arange(1134)] = 1.0
    c1oh = np.zeros((1134, 18), np.float32)
    c1oh[np.arange(1134), cols] = 1.0
    c, i, kd, kh, kw = np.meshgrid(
        np.arange(4), np.arange(2), np.arange(3), np.arange(3), np.arange(3),
        indexing="ij")
    rows2 = (kd * 18 + i * 9 + kh * 3 + kw).ravel()
    r2t = np.zeros((54, 216), np.float32)
    r2t[rows2, np.arange(216)] = 1.0
    c2oh = np.zeros((216, 4), np.float32)
    c2oh[np.arange(216), c.ravel()] = 1.0
    c, i, kd = np.meshgrid(np.arange(16), np.arange(4), np.arange(3),
                           indexing="ij")
    rows3 = (kd * 4 + i).ravel()
    r3t = np.zeros((12, 192), np.float32)
    r3t[rows3, np.arange(192)] = 1.0
    c3oh = np.zeros((192, 16), np.float32)
    c3oh[np.arange(192), c.ravel()] = 1.0
    b1sel = np.zeros((2, 18), np.float32)
    b1sel[np.repeat(np.arange(2), 9), np.arange(18)] = 1.0
    return e1, r1t, c1oh, r2t, c2oh, r3t, c3oh, b1sel


_E1, _R1T, _C1OH, _R2T, _C2OH, _R3T, _C3OH, _B1SEL = _placement_constants()
_HI = jax.lax.Precision.HIGHEST


def _weight_prep(c1wf, c1br, c2wf, c3wf, e1, r1t, c1oh, r2t, c2oh, r3t,
                 c3oh, b1sel):
    vals1 = jax.lax.dot_general(e1, c1wf, (((1,), (0,)), ((), ())))
    m1 = jax.lax.dot_general(r1t, vals1 * c1oh,
                             (((1,), (0,)), ((), ())))
    b1c = jax.lax.dot_general(c1br, b1sel, (((1,), (0,)), ((), ())),
                              precision=_HI)
    w2cat = jax.lax.dot_general(r2t, c2wf * c2oh,
                                (((1,), (0,)), ((), ())))
    w3cat = jax.lax.dot_general(r3t, c3wf * c3oh,
                                (((1,), (0,)), ((), ())))
    return m1, b1c, w2cat, w3cat


def _fused_kernel(x_ref, c1wf_ref, c1br_ref, c2wf_ref, cb2_ref,
                  c3wf_ref, cb3_ref, e1_ref, r1t_ref, c1oh_ref,
                  r2t_ref, c2oh_ref, r3t_ref, c3oh_ref, b1sel_ref,
                  w1_ref, w2_ref, g1_ref, gb1_ref, g2_ref, gb2_ref,
                  out_ref):
    m1, b1c, w2cat, w3cat = _weight_prep(
        c1wf_ref[...], c1br_ref[...], c2wf_ref[...], c3wf_ref[...],
        e1_ref[...], r1t_ref[...], c1oh_ref[...], r2t_ref[...],
        c2oh_ref[...], r3t_ref[...], c3oh_ref[...], b1sel_ref[...])

    xt = jnp.transpose(x_ref[...], (1, 0, 2))
    zpad = jnp.zeros((1, N, 25), jnp.float32)
    zpad3 = jnp.zeros((3, N, 25), jnp.float32)
    xpd = jnp.concatenate([zpad, xt, zpad3], axis=0)
    x1 = jnp.concatenate([xpd[k:k + DQ] for k in range(7)],
                         axis=2).reshape(RB, 175)
    h1 = jnp.dot(x1, m1, preferred_element_type=jnp.float32)
    h1 = jnp.maximum(h1 + b1c, 0.0)

    z4 = jnp.zeros((N, 4), jnp.float32)
    p0 = jnp.dot(h1, w2cat[0:18, :], preferred_element_type=jnp.float32)
    p1 = jnp.dot(h1, w2cat[18:36, :], preferred_element_type=jnp.float32)
    p2 = jnp.dot(h1, w2cat[36:54, :], preferred_element_type=jnp.float32)
    h2 = (jnp.concatenate([z4, p0[:-N, :]], axis=0) + p1
          + jnp.concatenate([p2[N:, :], z4], axis=0))
    h2 = jnp.maximum(h2 + cb2_ref[...], 0.0)

    z16 = jnp.zeros((N, 16), jnp.float32)
    q0 = jnp.dot(h2, w3cat[0:4, :], preferred_element_type=jnp.float32)
    q1 = jnp.dot(h2, w3cat[4:8, :], preferred_element_type=jnp.float32)
    q2 = jnp.dot(h2, w3cat[8:12, :], preferred_element_type=jnp.float32)
    h3 = (jnp.concatenate([z16, q0[:-N, :]], axis=0) + q1
          + jnp.concatenate([q2[N:, :], z16], axis=0))
    h3 = jnp.maximum(h3 + cb3_ref[...], 0.0)

    fr = jnp.concatenate([h3[dd * N:(dd + 1) * N, :] for dd in range(DQ)],
                         axis=1)

    gram = jax.lax.dot_general(
        fr, fr, (((1,), (1,)), ((), ())),
        preferred_element_type=jnp.float32,
        precision=jax.lax.Precision.HIGHEST)
    colid = jax.lax.broadcasted_iota(jnp.int32, (N, N), 1)
    rowid = jax.lax.broadcasted_iota(jnp.int32, (N, N), 0)
    diag = jnp.where(rowid == colid, gram, 0.0)
    sqn = jnp.sum(diag, axis=1, keepdims=True)
    sqn_t = jnp.sum(diag, axis=0, keepdims=True)
    sq = sqn + sqn_t - 2.0 * gram
    e = jnp.exp(sq * (-1.0 / 10.0))

    m = e
    selected = colid < 0
    for _ in range(8):
        rmax = jnp.max(m, axis=1, keepdims=True)
        first = jnp.min(jnp.where(m >= rmax, colid, N), axis=1, keepdims=True)
        sel = colid == first
        selected = jnp.logical_or(selected, sel)
        m = jnp.where(sel, -1.0, m)

    adj0 = jnp.where(selected, 0.9, 0.0)
    adj0 = jnp.where(rowid == colid, 0.5, adj0)
    eye = jnp.where(rowid == colid, 1.0, 0.0)
    adjacency = eye + adj0 * e
    invd = 1.0 / jnp.sqrt(jnp.sum(adjacency, axis=1, keepdims=True) + 1.0)
    dinv = jnp.where(rowid == colid, invd, 0.0)
    anorm = jnp.dot(jnp.dot(dinv, adjacency, preferred_element_type=jnp.float32),
                    dinv, preferred_element_type=jnp.float32)

    w1p = jnp.transpose(w1_ref[...].reshape(16, DQ, N),
                        (1, 0, 2)).reshape(DQ * 16, N)
    u1 = jnp.dot(fr, w1p, preferred_element_type=jnp.float32)
    s1 = jnp.dot(anorm, u1, preferred_element_type=jnp.float32)
    mu1 = jnp.mean(s1, axis=0, keepdims=True)
    xc1 = s1 - mu1
    var1 = jnp.mean(xc1 * xc1, axis=0, keepdims=True)
    s1 = jnp.tanh(g1_ref[...] * xc1 * jax.lax.rsqrt(var1 + 1e-5) + gb1_ref[...])

    u2 = jnp.dot(s1, w2_ref[...], preferred_element_type=jnp.float32)
    s2 = jnp.dot(anorm, u2, preferred_element_type=jnp.float32)
    mu2 = jnp.mean(s2, axis=0, keepdims=True)
    xc2 = s2 - mu2
    var2 = jnp.mean(xc2 * xc2, axis=0, keepdims=True)
    out_ref[...] = jnp.tanh(g2_ref[...] * xc2 * jax.lax.rsqrt(var2 + 1e-5)
                            + gb2_ref[...])


def kernel(x, c1w, c1b, c2w, c2b, c3w, c3b, w1, w2, g1, b1, g2, b2):
    ins = (x.reshape(N, 100, 25), c1w.reshape(126, 1), c1b[None, :],
           c2w.reshape(216, 1), c2b[None, :], c3w.reshape(192, 1),
           c3b[None, :], jnp.asarray(_E1), jnp.asarray(_R1T),
           jnp.asarray(_C1OH), jnp.asarray(_R2T), jnp.asarray(_C2OH),
           jnp.asarray(_R3T), jnp.asarray(_C3OH), jnp.asarray(_B1SEL),
           w1, w2, g1[None, :], b1[None, :], g2[None, :], b2[None, :])
    return pl.pallas_call(
        _fused_kernel,
        in_specs=[pl.BlockSpec(a.shape, lambda s=a.shape: (0,) * len(s))
                  for a in ins],
        out_specs=pl.BlockSpec((N, 16), lambda: (0, 0)),
        out_shape=jax.ShapeDtypeStruct((N, 16), jnp.float32),
    )(*ins)
